# Initial kernel scaffold; baseline (speedup 1.0000x reference)
import jax
import jax.numpy as jnp
from jax.experimental import pallas as pl


def kernel(input_ids, token_type_ids, table):
    raise NotImplementedError("write your pallas kernel here")


# speedup vs baseline: 7.2151x; 7.2151x over previous
import functools
import math

import jax
import jax.numpy as jnp
from jax import lax
from jax.experimental import pallas as pl
from jax.experimental.pallas import tpu as pltpu
from jax.experimental.pallas import tpu_sc as plsc

D_DIM = 128
EMB_SCALE = math.sqrt(float(D_DIM))


def kernel(input_ids, token_type_ids, table):
    B, L = input_ids.shape
    N = B * L
    ids_a = input_ids.reshape(N)
    ids_b = token_type_ids.reshape(N)

    info = plsc.get_sparse_core_info()
    NC, NS = info.num_cores, info.num_subcores
    NW = NC * NS
    assert N % NW == 0
    per_w = N // NW
    C = 256
    assert per_w % C == 0
    n_chunks = per_w // C

    mesh = plsc.VectorSubcoreMesh(core_axis_name="c", subcore_axis_name="s")

    @functools.partial(
        pl.kernel,
        mesh=mesh,
        out_type=jax.ShapeDtypeStruct((N, D_DIM), jnp.float32),
        scratch_types=[
            pltpu.VMEM((C,), jnp.int32),
            pltpu.VMEM((C,), jnp.int32),
            pltpu.VMEM((C, D_DIM), jnp.float32),
            pltpu.VMEM((C, D_DIM), jnp.float32),
            pltpu.SemaphoreType.DMA,
            pltpu.SemaphoreType.DMA,
        ],
    )
    def sc_embed(tab_hbm, a_hbm, b_hbm, out_hbm, idx_a, idx_b, buf_a, buf_b,
                 sem_a, sem_b):
        wid = lax.axis_index("s") * NC + lax.axis_index("c")
        base = wid * per_w

        def chunk_body(g, carry):
            off = base + g * C
            pltpu.sync_copy(a_hbm.at[pl.ds(off, C)], idx_a)
            pltpu.sync_copy(b_hbm.at[pl.ds(off, C)], idx_b)
            cp_a = pltpu.async_copy(tab_hbm.at[idx_a], buf_a, sem_a)
            cp_b = pltpu.async_copy(tab_hbm.at[idx_b], buf_b, sem_b)
            cp_a.wait()
            cp_b.wait()

            def row_body(r, carry2):
                for j in range(D_DIM // 16):
                    s = pl.ds(j * 16, 16)
                    buf_a[r, s] = buf_a[r, s] * EMB_SCALE + buf_b[r, s]
                return carry2

            lax.fori_loop(0, C, row_body, 0)
            pltpu.sync_copy(buf_a, out_hbm.at[pl.ds(off, C)])
            return carry

        lax.fori_loop(0, n_chunks, chunk_body, 0)

    out = sc_embed(table, ids_a, ids_b)
    return out.reshape(B, L, D_DIM)


# speedup vs baseline: 11.9240x; 1.6527x over previous
import functools
import math

import jax
import jax.numpy as jnp
from jax import lax
from jax.experimental import pallas as pl
from jax.experimental.pallas import tpu as pltpu
from jax.experimental.pallas import tpu_sc as plsc

D_DIM = 128
EMB_SCALE = math.sqrt(float(D_DIM))


def kernel(input_ids, token_type_ids, table):
    B, L = input_ids.shape
    N = B * L
    ids_a = input_ids.reshape(N)
    ids_b = token_type_ids.reshape(N)

    info = plsc.get_sparse_core_info()
    NC, NS = info.num_cores, info.num_subcores
    NW = NC * NS
    assert N % NW == 0
    per_w = N // NW
    C = 160
    assert per_w % (2 * C) == 0
    n_chunks = per_w // C
    H = n_chunks // 2

    mesh = plsc.VectorSubcoreMesh(core_axis_name="c", subcore_axis_name="s")

    @functools.partial(
        pl.kernel,
        mesh=mesh,
        out_type=jax.ShapeDtypeStruct((N, D_DIM), jnp.float32),
        scratch_types=[
            pltpu.VMEM((C,), jnp.int32),
            pltpu.VMEM((C,), jnp.int32),
            pltpu.VMEM((C,), jnp.int32),
            pltpu.VMEM((C,), jnp.int32),
            pltpu.VMEM((C, D_DIM), jnp.float32),
            pltpu.VMEM((C, D_DIM), jnp.float32),
            pltpu.VMEM((C, D_DIM), jnp.float32),
            pltpu.VMEM((C, D_DIM), jnp.float32),
            pltpu.VMEM((C, D_DIM), jnp.float32),
            pltpu.VMEM((C, D_DIM), jnp.float32),
            pltpu.SemaphoreType.DMA,
            pltpu.SemaphoreType.DMA,
            pltpu.SemaphoreType.DMA,
            pltpu.SemaphoreType.DMA,
        ],
    )
    def sc_embed(tab, a_hbm, b_hbm, out_hbm,
                 ia0, ib0, ia1, ib1,
                 ba0, bb0, bo0, ba1, bb1, bo1,
                 sg0, sg1, so0, so1):
        wid = lax.axis_index("s") * NC + lax.axis_index("c")
        base = wid * per_w

        def fetch(g, ia, ib, ba, bb, sg):
            off = base + g * C
            pltpu.sync_copy(a_hbm.at[pl.ds(off, C)], ia)
            pltpu.sync_copy(b_hbm.at[pl.ds(off, C)], ib)
            pltpu.async_copy(tab.at[ia], ba, sg)
            pltpu.async_copy(tab.at[ib], bb, sg)

        def wait_gathers(ia, ib, ba, bb, sg):
            pltpu.make_async_copy(tab.at[ia], ba, sg).wait()
            pltpu.make_async_copy(tab.at[ib], bb, sg).wait()

        def compute(ba, bb, bo):
            @plsc.parallel_loop(0, C, 1, unroll=2)
            def _(r):
                for j in range(D_DIM // 16):
                    s = pl.ds(j * 16, 16)
                    bo[r, s] = ba[r, s] * EMB_SCALE + bb[r, s]

        def put(g, bo, so):
            pltpu.async_copy(bo, out_hbm.at[pl.ds(base + g * C, C)], so)

        def wait_put(bo, so):
            pltpu.make_async_copy(bo, out_hbm.at[pl.ds(base, C)], so).wait()

        fetch(0, ia0, ib0, ba0, bb0, sg0)

        def body(h, carry):
            g0 = 2 * h
            fetch(g0 + 1, ia1, ib1, ba1, bb1, sg1)
            wait_gathers(ia0, ib0, ba0, bb0, sg0)

            @pl.when(h > 0)
            def _():
                wait_put(bo0, so0)

            compute(ba0, bb0, bo0)
            put(g0, bo0, so0)

            @pl.when(h < H - 1)
            def _():
                fetch(g0 + 2, ia0, ib0, ba0, bb0, sg0)

            wait_gathers(ia1, ib1, ba1, bb1, sg1)

            @pl.when(h > 0)
            def _():
                wait_put(bo1, so1)

            compute(ba1, bb1, bo1)
            put(g0 + 1, bo1, so1)
            return carry

        lax.fori_loop(0, H, body, 0)
        wait_put(bo0, so0)
        wait_put(bo1, so1)

    out = sc_embed(table, ids_a, ids_b)
    return out.reshape(B, L, D_DIM)
